# baseline (speedup 1.0000x reference)
import functools

import jax
import jax.numpy as jnp
from jax import lax
from jax.experimental import pallas as pl
from jax.experimental.pallas import tpu as pltpu
from jax.experimental.pallas import tpu_sc as plsc

NUM_FIELDS = 26
EMBED_DIM = 16
FIELD_SIZE = 100000

NC = 2
NS = 16
LANES = 16
ROWS_PER_BLK = 8
SUB = 128


@functools.partial(jax.jit, static_argnames=("batch",))
def _embedding_gather(xT, tab_v, *, batch):
    NW = NC * NS
    BB = batch // NW
    NSUB = BB // SUB
    TOT = NUM_FIELDS * BB

    mesh = plsc.VectorSubcoreMesh(core_axis_name="c", subcore_axis_name="s")

    @functools.partial(
        pl.kernel,
        out_type=jax.ShapeDtypeStruct((NUM_FIELDS, EMBED_DIM, batch), jnp.float32),
        mesh=mesh,
        scratch_types=[
            pltpu.VMEM((NUM_FIELDS, BB), jnp.int32),
            pltpu.VMEM((TOT,), jnp.int32),
            [pltpu.VMEM((SUB, 128), jnp.float32)] * 4,
            [pltpu.VMEM((EMBED_DIM, SUB), jnp.float32)] * 2,
            [pltpu.SemaphoreType.DMA] * 4,
            [pltpu.SemaphoreType.DMA] * 2,
        ],
        compiler_params=pltpu.CompilerParams(
            use_tc_tiling_on_sc=True, needs_layout_passes=False
        ),
    )
    def k(xT_hbm, tab_hbm, out_hbm, sub16s, blk_all, grows, outs, gsems, osems):
        wid = lax.axis_index("s") * NC + lax.axis_index("c")
        b0 = wid * BB

        pltpu.sync_copy(xT_hbm.at[:, pl.ds(b0, BB)], sub16s)

        def pre_body(j, _):
            s = j * LANES
            f = j // (BB // LANES)
            fr = j % (BB // LANES)
            v = sub16s[f, pl.ds(fr * LANES, LANES)] + f * FIELD_SIZE
            blk_all[pl.ds(s, LANES)] = lax.shift_right_logical(v, 3)
            sub16s[f, pl.ds(fr * LANES, LANES)] = lax.shift_left(
                lax.bitwise_and(v, 7), 4
            )
            return 0

        lax.fori_loop(0, TOT // LANES, pre_body, 0)

        def fire_gather(f, c):
            return pltpu.async_copy(
                tab_hbm.at[blk_all.at[pl.ds(f * BB + c * SUB, SUB)]],
                grows[c],
                gsems[c],
            )

        def extract(f, c):
            src = grows[c]
            dst = outs[c % 2]

            def body(j, _):
                rvec = j * LANES + lax.iota(jnp.int32, LANES)
                sub16 = sub16s[f, pl.ds(c * SUB + j * LANES, LANES)]
                for e in range(EMBED_DIM):
                    dst[e, pl.ds(j * LANES, LANES)] = plsc.load_gather(
                        src, [rvec, sub16 + e]
                    )
                return 0

            lax.fori_loop(0, SUB // LANES, body, 0)

        def fire_out(f, c):
            return pltpu.async_copy(
                outs[c % 2],
                out_hbm.at[f, :, pl.ds(b0 + c * SUB, SUB)],
                osems[c % 2],
            )

        def drain_out(p):
            pltpu.make_async_copy(
                out_hbm.at[0, :, pl.ds(b0, SUB)], outs[p], osems[p]
            ).wait()

        def f_body(f, _):
            gcopies = {c: fire_gather(f, c) for c in range(NSUB)}
            ocopies = {}
            for c in range(NSUB):
                gcopies.pop(c).wait()
                if c >= 2:
                    ocopies.pop(c - 2).wait()
                else:
                    @pl.when(f >= 1)
                    def _():
                        drain_out(c % 2)

                extract(f, c)
                ocopies[c] = fire_out(f, c)
            return 0

        lax.fori_loop(0, NUM_FIELDS, f_body, 0)
        drain_out(0)
        drain_out(1)

    return k(xT, tab_v)


BLK_LANES = 25600
BLK_COLS = BLK_LANES // 128


@jax.jit
def _repack_table(tableT):
    R = tableT.shape[1]
    nblk = (R + BLK_LANES - 1) // BLK_LANES

    def body(in_ref, out_ref):
        xt3 = in_ref[...].T.reshape(BLK_COLS * 16, ROWS_PER_BLK, EMBED_DIM)
        for j in range(ROWS_PER_BLK):
            out_ref[:, j * EMBED_DIM:(j + 1) * EMBED_DIM] = xt3[:, j, :]

    return pl.pallas_call(
        body,
        grid=(nblk,),
        in_specs=[
            pl.BlockSpec((EMBED_DIM, BLK_LANES), lambda i: (0, i)),
        ],
        out_specs=pl.BlockSpec((BLK_COLS * 16, 128), lambda i: (i, 0)),
        out_shape=jax.ShapeDtypeStruct((R // ROWS_PER_BLK, 128), jnp.float32),
    )(tableT)


def kernel(x, table):
    batch = x.shape[0]
    tab_v = _repack_table(table.T)
    out_view = _embedding_gather(x.T, tab_v, batch=batch)
    return jnp.transpose(out_view, (2, 0, 1))


# speedup vs baseline: 1.0588x; 1.0588x over previous
import functools

import jax
import jax.numpy as jnp
from jax import lax
from jax.experimental import pallas as pl
from jax.experimental.pallas import tpu as pltpu
from jax.experimental.pallas import tpu_sc as plsc

NUM_FIELDS = 26
EMBED_DIM = 16
FIELD_SIZE = 100000

NC = 2
NS = 16
LANES = 16
ROWS_PER_BLK = 8
SUB = 128


@functools.partial(jax.jit, static_argnames=("batch",))
def _embedding_gather(xT, tab, *, batch):
    NW = NC * NS
    BB = batch // NW
    NSUB = BB // SUB
    TOT = NUM_FIELDS * BB

    mesh = plsc.VectorSubcoreMesh(core_axis_name="c", subcore_axis_name="s")

    @functools.partial(
        pl.kernel,
        out_type=jax.ShapeDtypeStruct((NUM_FIELDS, EMBED_DIM, batch), jnp.float32),
        mesh=mesh,
        scratch_types=[
            pltpu.VMEM((NUM_FIELDS, BB), jnp.int32),
            pltpu.VMEM((TOT,), jnp.int32),
            [pltpu.VMEM((SUB, EMBED_DIM), jnp.float32)] * 2,
            [pltpu.VMEM((EMBED_DIM, SUB), jnp.float32)] * 2,
            [pltpu.SemaphoreType.DMA] * 2,
            [pltpu.SemaphoreType.DMA] * 2,
        ],
        compiler_params=pltpu.CompilerParams(
            use_tc_tiling_on_sc=False, needs_layout_passes=False
        ),
    )
    def k(xT_hbm, tab_hbm, out_hbm, xstage, idx_all, grows, outs, gsems, osems):
        wid = lax.axis_index("s") * NC + lax.axis_index("c")
        b0 = wid * BB

        pltpu.sync_copy(xT_hbm.at[:, pl.ds(b0, BB)], xstage)

        def pre_body(j, _):
            s = j * LANES
            f = j // (BB // LANES)
            fr = j % (BB // LANES)
            idx_all[pl.ds(s, LANES)] = (
                xstage[f, pl.ds(fr * LANES, LANES)] + f * FIELD_SIZE
            )
            return 0

        lax.fori_loop(0, TOT // LANES, pre_body, 0)

        def fire_gather(f, c):
            return pltpu.async_copy(
                tab_hbm.at[idx_all.at[pl.ds(f * BB + c * SUB, SUB)]],
                grows[c % 2],
                gsems[c % 2],
            )

        def extract(f, c):
            src = grows[c % 2]
            dst = outs[c % 2]

            def body(j, _):
                rvec = j * LANES + lax.iota(jnp.int32, LANES)
                for e in range(EMBED_DIM):
                    cvec = jnp.full((LANES,), e, dtype=jnp.int32)
                    dst[e, pl.ds(j * LANES, LANES)] = plsc.load_gather(
                        src, [rvec, cvec]
                    )
                return 0

            lax.fori_loop(0, SUB // LANES, body, 0)

        def fire_out(f, c):
            return pltpu.async_copy(
                outs[c % 2],
                out_hbm.at[f, :, pl.ds(b0 + c * SUB, SUB)],
                osems[c % 2],
            )

        def drain_out(p):
            pltpu.make_async_copy(
                out_hbm.at[0, :, pl.ds(b0, SUB)], outs[p], osems[p]
            ).wait()

        def f_body(f, _):
            gcopies = {0: fire_gather(f, 0)}
            ocopies = {}
            for c in range(NSUB):
                if c + 1 < NSUB:
                    gcopies[c + 1] = fire_gather(f, c + 1)
                gcopies.pop(c).wait()
                if c >= 2:
                    ocopies.pop(c - 2).wait()
                else:
                    @pl.when(f >= 1)
                    def _():
                        drain_out(c % 2)

                extract(f, c)
                ocopies[c] = fire_out(f, c)
            return 0

        lax.fori_loop(0, NUM_FIELDS, f_body, 0)
        drain_out(0)
        drain_out(1)

    return k(xT, tab)


BLK_LANES = 25600
BLK_COLS = BLK_LANES // 128


@jax.jit
def _repack_table(tableT):
    R = tableT.shape[1]
    nblk = (R + BLK_LANES - 1) // BLK_LANES

    def body(in_ref, out_ref):
        xt3 = in_ref[...].T.reshape(BLK_COLS * 16, ROWS_PER_BLK, EMBED_DIM)
        for j in range(ROWS_PER_BLK):
            out_ref[:, j * EMBED_DIM:(j + 1) * EMBED_DIM] = xt3[:, j, :]

    return pl.pallas_call(
        body,
        grid=(nblk,),
        in_specs=[
            pl.BlockSpec((EMBED_DIM, BLK_LANES), lambda i: (0, i)),
        ],
        out_specs=pl.BlockSpec((BLK_COLS * 16, 128), lambda i: (i, 0)),
        out_shape=jax.ShapeDtypeStruct((R // ROWS_PER_BLK, 128), jnp.float32),
    )(tableT)


def kernel(x, table):
    batch = x.shape[0]
    tab = _repack_table(table.T).reshape(table.shape[0], EMBED_DIM)
    out_view = _embedding_gather(x.T, tab, batch=batch)
    return jnp.transpose(out_view, (2, 0, 1))


# speedup vs baseline: 1.0604x; 1.0015x over previous
import functools

import jax
import jax.numpy as jnp
from jax import lax
from jax.experimental import pallas as pl
from jax.experimental.pallas import tpu as pltpu
from jax.experimental.pallas import tpu_sc as plsc

NUM_FIELDS = 26
EMBED_DIM = 16
FIELD_SIZE = 100000

NC = 2
NS = 16
LANES = 16
ROWS_PER_BLK = 8
SUB = 128


@functools.partial(jax.jit, static_argnames=("batch",))
def _embedding_gather(xT, tab, *, batch):
    NW = NC * NS
    BB = batch // NW
    NSUB = BB // SUB
    TOT = NUM_FIELDS * BB

    mesh = plsc.VectorSubcoreMesh(core_axis_name="c", subcore_axis_name="s")

    @functools.partial(
        pl.kernel,
        out_type=jax.ShapeDtypeStruct((NUM_FIELDS, EMBED_DIM, batch), jnp.float32),
        mesh=mesh,
        scratch_types=[
            pltpu.VMEM((NUM_FIELDS, BB), jnp.int32),
            pltpu.VMEM((TOT,), jnp.int32),
            [pltpu.VMEM((SUB, EMBED_DIM), jnp.float32)] * 2,
            [pltpu.VMEM((EMBED_DIM, SUB), jnp.float32)] * 2,
            [pltpu.SemaphoreType.DMA] * 2,
            [pltpu.SemaphoreType.DMA] * 2,
        ],
        compiler_params=pltpu.CompilerParams(
            use_tc_tiling_on_sc=False, needs_layout_passes=False
        ),
    )
    def k(xT_hbm, tab_hbm, out_hbm, xstage, idx_all, grows, outs, gsems, osems):
        wid = lax.axis_index("s") * NC + lax.axis_index("c")
        b0 = wid * BB

        pltpu.sync_copy(xT_hbm.at[:, pl.ds(b0, BB)], xstage)

        def pre_body(j, _):
            s = j * LANES
            f = j // (BB // LANES)
            fr = j % (BB // LANES)
            idx_all[pl.ds(s, LANES)] = (
                xstage[f, pl.ds(fr * LANES, LANES)] + f * FIELD_SIZE
            )
            return 0

        lax.fori_loop(0, TOT // LANES, pre_body, 0)

        def fire_gather(f, c):
            return pltpu.async_copy(
                tab_hbm.at[idx_all.at[pl.ds(f * BB + c * SUB, SUB)]],
                grows[c % 2],
                gsems[c % 2],
            )

        def extract(f, c):
            src = grows[c % 2]
            dst = outs[c % 2]

            def body(j, _):
                rvec = j * LANES + lax.iota(jnp.int32, LANES)
                for e in range(EMBED_DIM):
                    cvec = jnp.full((LANES,), e, dtype=jnp.int32)
                    dst[e, pl.ds(j * LANES, LANES)] = plsc.load_gather(
                        src, [rvec, cvec]
                    )
                return 0

            lax.fori_loop(0, SUB // LANES, body, 0)

        def fire_out(f, c):
            return pltpu.async_copy(
                outs[c % 2],
                out_hbm.at[f, :, pl.ds(b0 + c * SUB, SUB)],
                osems[c % 2],
            )

        def drain_out(p):
            pltpu.make_async_copy(
                out_hbm.at[0, :, pl.ds(b0, SUB)], outs[p], osems[p]
            ).wait()

        def f_body(f, _):
            gcopies = {0: fire_gather(f, 0)}
            ocopies = {}
            for c in range(NSUB):
                if c + 1 < NSUB:
                    gcopies[c + 1] = fire_gather(f, c + 1)
                gcopies.pop(c).wait()
                if c >= 2:
                    ocopies.pop(c - 2).wait()
                else:
                    @pl.when(f >= 1)
                    def _():
                        drain_out(c % 2)

                extract(f, c)
                ocopies[c] = fire_out(f, c)
            return 0

        lax.fori_loop(0, NUM_FIELDS, f_body, 0)
        drain_out(0)
        drain_out(1)

    return k(xT, tab)


BLK_LANES = 51200
BLK_COLS = BLK_LANES // 128


@jax.jit
def _repack_table(tableT):
    R = tableT.shape[1]
    nblk = (R + BLK_LANES - 1) // BLK_LANES

    def body(in_ref, out_ref):
        xt3 = in_ref[...].T.reshape(BLK_COLS * 16, ROWS_PER_BLK, EMBED_DIM)
        for j in range(ROWS_PER_BLK):
            out_ref[:, j * EMBED_DIM:(j + 1) * EMBED_DIM] = xt3[:, j, :]

    return pl.pallas_call(
        body,
        grid=(nblk,),
        in_specs=[
            pl.BlockSpec((EMBED_DIM, BLK_LANES), lambda i: (0, i)),
        ],
        out_specs=pl.BlockSpec((BLK_COLS * 16, 128), lambda i: (i, 0)),
        out_shape=jax.ShapeDtypeStruct((R // ROWS_PER_BLK, 128), jnp.float32),
    )(tableT)


def kernel(x, table):
    batch = x.shape[0]
    tab = _repack_table(table.T).reshape(table.shape[0], EMBED_DIM)
    out_view = _embedding_gather(x.T, tab, batch=batch)
    return jnp.transpose(out_view, (2, 0, 1))
